# Initial kernel scaffold; baseline (speedup 1.0000x reference)
import jax
import jax.numpy as jnp
from jax.experimental import pallas as pl


def kernel(points, scales):
    raise NotImplementedError("write your pallas kernel here")


# speedup vs baseline: 42.3231x; 42.3231x over previous
import functools

import jax
import jax.numpy as jnp
from jax import lax
from jax.experimental import pallas as pl
from jax.experimental.pallas import tpu as pltpu
from jax.experimental.pallas import tpu_sc as plsc

_N = 4096
_NW = 32
_RPW = _N // _NW
_L = 16
_CHUNKS = _N // _L
_UNROLL = 4


def _sqrt16(x):
    i = plsc.bitcast(x, jnp.int32)
    i = jnp.int32(0x5F3759DF) - (i >> 1)
    y = plsc.bitcast(i, jnp.float32)
    xh = x * jnp.float32(0.5)
    for _ in range(3):
        y = y * (jnp.float32(1.5) - xh * y * y)
    return x * y


def _knn_body(pts_hbm, scl_hbm, out_hbm, pts_v, scl_v, s_v, out_v):
    cid = lax.axis_index("c")
    sid = lax.axis_index("s")
    wid = sid * 2 + cid
    base = wid * _RPW

    pltpu.sync_copy(pts_hbm, pts_v)
    for k in range(3):
        pltpu.sync_copy(scl_hbm.at[pl.ds(k * _N + base, _RPW)],
                        scl_v.at[pl.ds(k * _RPW, _RPW)])

    inf = jnp.float32(jnp.inf)
    lanes = lax.iota(jnp.int32, 16)
    zeros16 = jnp.zeros((_L,), jnp.int32)
    lane0 = lanes == 0

    def q_body(q, carry_q):
        qiv = jnp.full((_L,), base + q, jnp.int32)
        qx = plsc.load_gather(pts_v, [qiv])
        qy = plsc.load_gather(pts_v, [qiv + _N])
        qz = plsc.load_gather(pts_v, [qiv + 2 * _N])

        def c_body(j, carry):
            a, b, c, e = carry
            for u in range(_UNROLL):
                off = (j * _UNROLL + u) * _L
                cx = pts_v[pl.ds(off, _L)]
                cy = pts_v[pl.ds(_N + off, _L)]
                cz = pts_v[pl.ds(2 * _N + off, _L)]
                dx = cx - qx
                dy = cy - qy
                dz = cz - qz
                d = dx * dx + dy * dy + dz * dz
                a2 = jnp.minimum(a, d)
                t = jnp.maximum(a, d)
                b2 = jnp.minimum(b, t)
                t = jnp.maximum(b, t)
                c2 = jnp.minimum(c, t)
                t = jnp.maximum(c, t)
                e2 = jnp.minimum(e, t)
                a, b, c, e = a2, b2, c2, e2
            return a, b, c, e

        init = (jnp.full((_L,), inf), jnp.full((_L,), inf),
                jnp.full((_L,), inf), jnp.full((_L,), inf))
        a, b, c, e = lax.fori_loop(0, _CHUNKS // _UNROLL, c_body, init)

        qv = jnp.full((_L,), q, jnp.int32)
        for r in range(4):
            m = jnp.min(a)
            if r > 0:
                plsc.store_scatter(s_v, [qv + (r - 1) * _RPW],
                                   jnp.full((_L,), m, jnp.float32),
                                   mask=lane0)
            sel = lanes == plsc.all_reduce_ffs(a == m)
            a = jnp.where(sel, b, a)
            b = jnp.where(sel, c, b)
            c = jnp.where(sel, e, c)
            e = jnp.where(sel, inf, e)
        return carry_q

    lax.fori_loop(0, _RPW, q_body, 0)

    third = jnp.float32(1.0 / 3.0)

    def f_body(v, carry_f):
        o = v * _L
        r = (_sqrt16(s_v[pl.ds(o, _L)]) + _sqrt16(s_v[pl.ds(_RPW + o, _L)])
             + _sqrt16(s_v[pl.ds(2 * _RPW + o, _L)])) * third
        r = jnp.maximum(r, jnp.float32(1e-5))
        for k in range(3):
            out_v[pl.ds(k * _RPW + o, _L)] = scl_v[pl.ds(k * _RPW + o, _L)] * r
        return carry_f

    lax.fori_loop(0, _RPW // _L, f_body, 0)

    for k in range(3):
        pltpu.sync_copy(out_v.at[pl.ds(k * _RPW, _RPW)],
                        out_hbm.at[pl.ds(k * _N + base, _RPW)])


_knn = functools.partial(
    pl.kernel,
    mesh=plsc.VectorSubcoreMesh(core_axis_name="c", subcore_axis_name="s"),
    compiler_params=pltpu.CompilerParams(needs_layout_passes=False),
    out_type=jax.ShapeDtypeStruct((3 * _N,), jnp.float32),
    scratch_types=[
        pltpu.VMEM((3 * _N,), jnp.float32),
        pltpu.VMEM((3 * _RPW,), jnp.float32),
        pltpu.VMEM((3 * _RPW,), jnp.float32),
        pltpu.VMEM((3 * _RPW,), jnp.float32),
    ],
)(_knn_body)


def kernel(points, scales):
    out_flat = _knn(points.T.reshape(-1), scales.T.reshape(-1))
    return out_flat.reshape(3, _N).T


# speedup vs baseline: 48.9521x; 1.1566x over previous
import functools

import jax
import jax.numpy as jnp
from jax import lax
from jax.experimental import pallas as pl
from jax.experimental.pallas import tpu as pltpu
from jax.experimental.pallas import tpu_sc as plsc

_N = 4096
_NW = 32
_RPW = _N // _NW
_L = 16
_CHUNKS = _N // _L
_UNROLL = 4


def _sqrt16(x):
    i = plsc.bitcast(x, jnp.int32)
    i = jnp.int32(0x5F3759DF) - (i >> 1)
    y = plsc.bitcast(i, jnp.float32)
    xh = x * jnp.float32(0.5)
    for _ in range(3):
        y = y * (jnp.float32(1.5) - xh * y * y)
    return x * y


def _knn_body(pts_hbm, scl_hbm, out_hbm, pts_v, sq_v, scl_v, s_v, out_v):
    cid = lax.axis_index("c")
    sid = lax.axis_index("s")
    wid = sid * 2 + cid
    base = wid * _RPW

    pltpu.sync_copy(pts_hbm, pts_v)
    for k in range(3):
        pltpu.sync_copy(scl_hbm.at[pl.ds(k * _N + base, _RPW)],
                        scl_v.at[pl.ds(k * _RPW, _RPW)])

    inf = jnp.float32(jnp.inf)
    lanes = lax.iota(jnp.int32, 16)
    zeros16 = jnp.zeros((_L,), jnp.int32)
    lane0 = lanes == 0
    infv = jnp.full((_L,), inf)

    def n_body(j, carry_n):
        off = j * _L
        cx = pts_v[pl.ds(off, _L)]
        cy = pts_v[pl.ds(_N + off, _L)]
        cz = pts_v[pl.ds(2 * _N + off, _L)]
        sq_v[pl.ds(off, _L)] = cx * cx + cy * cy + cz * cz
        return carry_n

    lax.fori_loop(0, _CHUNKS, n_body, 0)

    def q_body(q, carry_q):
        qiv = jnp.full((_L,), base + q, jnp.int32)
        mx = plsc.load_gather(pts_v, [qiv]) * jnp.float32(-2.0)
        my = plsc.load_gather(pts_v, [qiv + _N]) * jnp.float32(-2.0)
        mz = plsc.load_gather(pts_v, [qiv + 2 * _N]) * jnp.float32(-2.0)
        ppv = plsc.load_gather(sq_v, [qiv])
        plsc.store_scatter(sq_v, [qiv], infv, mask=lane0)

        def c_body(j, carry):
            a, b, c = carry
            for u in range(_UNROLL):
                off = (j * _UNROLL + u) * _L
                cx = pts_v[pl.ds(off, _L)]
                cy = pts_v[pl.ds(_N + off, _L)]
                cz = pts_v[pl.ds(2 * _N + off, _L)]
                d = sq_v[pl.ds(off, _L)] + ppv
                d = d + cx * mx
                d = d + cy * my
                d = d + cz * mz
                a2 = jnp.minimum(a, d)
                t = jnp.maximum(a, d)
                b2 = jnp.minimum(b, t)
                t = jnp.maximum(b, t)
                c2 = jnp.minimum(c, t)
                a, b, c = a2, b2, c2
            return a, b, c

        init = (infv, infv, infv)
        a, b, c = lax.fori_loop(0, _CHUNKS // _UNROLL, c_body, init)

        plsc.store_scatter(sq_v, [qiv], ppv, mask=lane0)

        qv = jnp.full((_L,), q, jnp.int32)
        for r in range(3):
            m = jnp.min(a)
            plsc.store_scatter(s_v, [qv + r * _RPW],
                               jnp.full((_L,), m, jnp.float32),
                               mask=lane0)
            sel = lanes == plsc.all_reduce_ffs(a == m)
            a = jnp.where(sel, b, a)
            b = jnp.where(sel, c, b)
            c = jnp.where(sel, inf, c)
        return carry_q

    lax.fori_loop(0, _RPW, q_body, 0)

    third = jnp.float32(1.0 / 3.0)

    zero = jnp.float32(0.0)

    def f_body(v, carry_f):
        o = v * _L
        r = (_sqrt16(jnp.maximum(s_v[pl.ds(o, _L)], zero))
             + _sqrt16(jnp.maximum(s_v[pl.ds(_RPW + o, _L)], zero))
             + _sqrt16(jnp.maximum(s_v[pl.ds(2 * _RPW + o, _L)], zero))) * third
        r = jnp.maximum(r, jnp.float32(1e-5))
        for k in range(3):
            out_v[pl.ds(k * _RPW + o, _L)] = scl_v[pl.ds(k * _RPW + o, _L)] * r
        return carry_f

    lax.fori_loop(0, _RPW // _L, f_body, 0)

    for k in range(3):
        pltpu.sync_copy(out_v.at[pl.ds(k * _RPW, _RPW)],
                        out_hbm.at[pl.ds(k * _N + base, _RPW)])


_knn = functools.partial(
    pl.kernel,
    mesh=plsc.VectorSubcoreMesh(core_axis_name="c", subcore_axis_name="s"),
    compiler_params=pltpu.CompilerParams(needs_layout_passes=False),
    out_type=jax.ShapeDtypeStruct((3 * _N,), jnp.float32),
    scratch_types=[
        pltpu.VMEM((3 * _N,), jnp.float32),
        pltpu.VMEM((_N,), jnp.float32),
        pltpu.VMEM((3 * _RPW,), jnp.float32),
        pltpu.VMEM((3 * _RPW,), jnp.float32),
        pltpu.VMEM((3 * _RPW,), jnp.float32),
    ],
)(_knn_body)


def kernel(points, scales):
    out_flat = _knn(points.T.reshape(-1), scales.T.reshape(-1))
    return out_flat.reshape(3, _N).T
